# baseline (speedup 1.0000x reference)
import functools

import jax
import jax.numpy as jnp
from jax import lax
from jax.experimental import pallas as pl
from jax.experimental.pallas import tpu as pltpu
from jax.experimental.pallas import tpu_sc as plsc

_L = 16


def _make_pe_add(B, T, D):
    info = plsc.get_sparse_core_info()
    NC, NS = info.num_cores, info.num_subcores
    NW = NC * NS
    TW = T // NW
    CH = 4
    NCH = TW // CH
    NBLK = D // _L

    mesh = plsc.VectorSubcoreMesh(core_axis_name="c", subcore_axis_name="s")

    @functools.partial(
        pl.kernel,
        out_type=jax.ShapeDtypeStruct((B, T, D), jnp.float32),
        mesh=mesh,
        scratch_types=(
            [pltpu.VMEM((B, CH, D), jnp.float32)] * 4
            + [pltpu.VMEM((CH, D), jnp.float32)] * 2
            + [pltpu.SemaphoreType.DMA] * 6
        ),
    )
    def pe_add(x_hbm, pe_hbm, out_hbm, *bufs):
        ins = bufs[0:2]
        outs = bufs[2:4]
        pes = bufs[4:6]
        sin = bufs[6:8]
        sout = bufs[8:10]
        spe = bufs[10:12]

        wid = lax.axis_index("s") * NC + lax.axis_index("c")
        t_base = wid * TW

        def x_in(c, s):
            return pltpu.make_async_copy(
                x_hbm.at[:, pl.ds(t_base + c * CH, CH), :], ins[s], sin[s])

        def pe_copy(c, s):
            return pltpu.make_async_copy(
                pe_hbm.at[pl.ds(t_base + c * CH, CH), :], pes[s], spe[s])

        def x_out(c, s):
            return pltpu.make_async_copy(
                outs[s], out_hbm.at[:, pl.ds(t_base + c * CH, CH), :],
                sout[s])

        for s in range(2):
            pe_copy(s, s).start()
            x_in(s, s).start()

        def pair_body(g, carry):
            for j in range(2):
                c = 2 * g + j
                s = j
                x_in(c, s).wait()
                pe_copy(c, s).wait()
                @pl.when(g > 0)
                def _():
                    x_out(c - 2, s).wait()

                ib, ob, pb = ins[s], outs[s], pes[s]

                def row_body(r, rc):
                    for blk in range(NBLK):
                        sl = pl.ds(blk * _L, _L)
                        pv = pb[r, sl]
                        for b in range(B):
                            ob[b, r, sl] = ib[b, r, sl] + pv
                    return rc

                lax.fori_loop(0, CH, row_body, 0)
                x_out(c, s).start()
                @pl.when(c + 2 < NCH)
                def _():
                    pe_copy(c + 2, s).start()
                    x_in(c + 2, s).start()
            return carry

        lax.fori_loop(0, NCH // 2, pair_body, 0)

        for s in range(2):
            x_out(NCH - 2 + s, s).wait()

    return pe_add


def kernel(x, posem):
    B, T, D = x.shape
    pe_add = _make_pe_add(B, T, D)
    return pe_add(x, posem)


# speedup vs baseline: 1.7963x; 1.7963x over previous
import functools

import jax
import jax.numpy as jnp
from jax import lax
from jax.experimental import pallas as pl
from jax.experimental.pallas import tpu as pltpu
from jax.experimental.pallas import tpu_sc as plsc

_L = 16


def _make_pe_add(B, T, D):
    info = plsc.get_sparse_core_info()
    NC, NS = info.num_cores, info.num_subcores
    NW = NC * NS
    TW = T // NW
    CH = 4
    NCH = TW // CH
    NBLK = D // _L

    mesh = plsc.VectorSubcoreMesh(core_axis_name="c", subcore_axis_name="s")

    @functools.partial(
        pl.kernel,
        out_type=jax.ShapeDtypeStruct((B, T, D), jnp.float32),
        mesh=mesh,
        scratch_types=(
            [pltpu.VMEM((CH, D), jnp.float32)] * (4 * B + 2)
            + [pltpu.SemaphoreType.DMA] * 6
        ),
    )
    def pe_add(x_hbm, pe_hbm, out_hbm, *bufs):
        ins = (bufs[0:B], bufs[B:2 * B])
        outs = (bufs[2 * B:3 * B], bufs[3 * B:4 * B])
        pes = bufs[4 * B:4 * B + 2]
        sin = bufs[4 * B + 2:4 * B + 4]
        sout = bufs[4 * B + 4:4 * B + 6]
        spe = bufs[4 * B + 6:4 * B + 8]

        wid = lax.axis_index("s") * NC + lax.axis_index("c")
        t_base = wid * TW

        def x_in(c, b, s):
            return pltpu.make_async_copy(
                x_hbm.at[b, pl.ds(t_base + c * CH, CH), :], ins[s][b], sin[s])

        def pe_copy(c, s):
            return pltpu.make_async_copy(
                pe_hbm.at[pl.ds(t_base + c * CH, CH), :], pes[s], spe[s])

        def x_out(c, b, s):
            return pltpu.make_async_copy(
                outs[s][b], out_hbm.at[b, pl.ds(t_base + c * CH, CH), :],
                sout[s])

        for s in range(2):
            pe_copy(s, s).start()
            for b in range(B):
                x_in(s, b, s).start()

        def pair_body(g, carry):
            for j in range(2):
                c = 2 * g + j
                s = j
                for b in range(B):
                    x_in(c, b, s).wait()
                pe_copy(c, s).wait()
                @pl.when(g > 0)
                def _():
                    for b in range(B):
                        x_out(c - 2, b, s).wait()

                ib, ob, pb = ins[s], outs[s], pes[s]

                @plsc.parallel_loop(0, CH * NBLK, unroll=4)
                def _(i):
                    r = i // NBLK
                    sl = pl.ds((i % NBLK) * _L, _L)
                    pv = pb[r, sl]
                    for b in range(B):
                        ob[b][r, sl] = ib[b][r, sl] + pv
                for b in range(B):
                    x_out(c, b, s).start()
                @pl.when(c + 2 < NCH)
                def _():
                    pe_copy(c + 2, s).start()
                    for b in range(B):
                        x_in(c + 2, b, s).start()
            return carry

        lax.fori_loop(0, NCH // 2, pair_body, 0)

        for s in range(2):
            for b in range(B):
                x_out(NCH - 2 + s, b, s).wait()

    return pe_add


def kernel(x, posem):
    B, T, D = x.shape
    pe_add = _make_pe_add(B, T, D)
    return pe_add(x, posem)
